# baseline (speedup 1.0000x reference)
import functools

import jax
import jax.numpy as jnp
from jax import lax
from jax.experimental import pallas as pl
from jax.experimental.pallas import tpu as pltpu
from jax.experimental.pallas import tpu_sc as plsc

N_CORES = 2
N_SUBCORES = 16
NW = N_CORES * N_SUBCORES
LANES = 16

B = 16384
D = 32
BPW = B // NW
CHUNK = 128
NCHUNK = BPW // CHUNK
GPC = CHUNK // LANES

UF_ROWS = 250000
BIAS_ROWS = 7813


def _mf_body(users_hbm, items_hbm, uf_hbm, if_hbm, ub_hbm, ib_hbm, gb_hbm,
             out_hbm, uidx_v, iidx_v, urow_v, irow_v, ubrow_v, ibrow_v,
             ufat_v, ifat_v, ubias_v, ibias_v, gb_v, out_v, fsem, bsem):
    wid = lax.axis_index("s") * N_CORES + lax.axis_index("c")
    base = wid * BPW
    pltpu.sync_copy(users_hbm.at[pl.ds(base, BPW)], uidx_v)
    pltpu.sync_copy(items_hbm.at[pl.ds(base, BPW)], iidx_v)

    gb_v[...] = jnp.zeros((LANES,), jnp.float32)
    pltpu.sync_copy(gb_hbm, gb_v.at[pl.ds(0, 1)])

    lanes = lax.iota(jnp.int32, LANES)

    for c in range(NCHUNK):
        csplat = jnp.full((LANES,), c, jnp.int32)
        for k in range(GPC):
            src = lanes + (c * CHUNK + k * LANES)
            dst = lanes + k * LANES
            u = plsc.load_gather(uidx_v, [src])
            i = plsc.load_gather(iidx_v, [src])
            plsc.store_scatter(urow_v, [csplat, dst], u >> 2)
            plsc.store_scatter(irow_v, [csplat, dst], i >> 2)
            plsc.store_scatter(ubrow_v, [csplat, dst], u >> 7)
            plsc.store_scatter(ibrow_v, [csplat, dst], i >> 7)

    gbs = jnp.sum(gb_v[...])

    def fire_factors(c, slot):
        return (
            pltpu.async_copy(uf_hbm.at[urow_v.at[c]], ufat_v.at[slot], fsem),
            pltpu.async_copy(if_hbm.at[irow_v.at[c]], ifat_v.at[slot], fsem),
        )

    def fire_biases(c):
        return (
            pltpu.async_copy(ub_hbm.at[ubrow_v.at[c]], ubias_v, bsem),
            pltpu.async_copy(ib_hbm.at[ibrow_v.at[c]], ibias_v, bsem),
        )

    inflight_f = fire_factors(0, 0)
    inflight_b = fire_biases(0)

    for c in range(NCHUNK):
        for cp in inflight_f:
            cp.wait()
        if c + 1 < NCHUNK:
            next_f = fire_factors(c + 1, (c + 1) % 2)
        else:
            next_f = ()
        for cp in inflight_b:
            cp.wait()
        slot = c % 2
        srow = jnp.full((LANES,), slot, jnp.int32)
        for g in range(GPC):
            e_in_chunk = lanes + g * LANES
            src = e_in_chunk + c * CHUNK
            u = plsc.load_gather(uidx_v, [src])
            i = plsc.load_gather(iidx_v, [src])
            ucol = (u & 3) << 5
            icol = (i & 3) << 5
            acc = (plsc.load_gather(ubias_v, [e_in_chunk, u & 127])
                   + plsc.load_gather(ibias_v, [e_in_chunk, i & 127]) + gbs)
            for j in range(D):
                uv = plsc.load_gather(ufat_v, [srow, e_in_chunk, ucol + j])
                iv = plsc.load_gather(ifat_v, [srow, e_in_chunk, icol + j])
                acc = acc + uv * iv
            plsc.store_scatter(out_v, [src], acc)
        if c + 1 < NCHUNK:
            inflight_b = fire_biases(c + 1)
        inflight_f = next_f

    pltpu.sync_copy(out_v, out_hbm.at[pl.ds(base, BPW)])


@functools.partial(
    pl.kernel,
    out_type=jax.ShapeDtypeStruct((B,), jnp.float32),
    mesh=plsc.VectorSubcoreMesh(core_axis_name="c", subcore_axis_name="s"),
    compiler_params=pltpu.CompilerParams(needs_layout_passes=False),
    scratch_types=[
        pltpu.VMEM((BPW,), jnp.int32),
        pltpu.VMEM((BPW,), jnp.int32),
        pltpu.VMEM((NCHUNK, CHUNK), jnp.int32),
        pltpu.VMEM((NCHUNK, CHUNK), jnp.int32),
        pltpu.VMEM((NCHUNK, CHUNK), jnp.int32),
        pltpu.VMEM((NCHUNK, CHUNK), jnp.int32),
        pltpu.VMEM((2, CHUNK, 128), jnp.float32),
        pltpu.VMEM((2, CHUNK, 128), jnp.float32),
        pltpu.VMEM((CHUNK, 128), jnp.float32),
        pltpu.VMEM((CHUNK, 128), jnp.float32),
        pltpu.VMEM((LANES,), jnp.float32),
        pltpu.VMEM((BPW,), jnp.float32),
        pltpu.SemaphoreType.DMA,
        pltpu.SemaphoreType.DMA,
    ],
)
def _mf_kernel(*refs):
    _mf_body(*refs)


def _reshape_table(tbl, gb):
    return tbl.reshape(UF_ROWS, 128) + 0.0 * gb[0]


def kernel(data, user_factors, item_factors, user_bias, item_bias, global_bias):
    users = data[:, 0]
    items = data[:, 1]
    uf4 = _reshape_table(user_factors, global_bias)
    if4 = _reshape_table(item_factors, global_bias)
    ubp = jnp.pad(user_bias[:, 0], (0, BIAS_ROWS * 128 - user_bias.shape[0]))
    ibp = jnp.pad(item_bias[:, 0], (0, BIAS_ROWS * 128 - item_bias.shape[0]))
    ub2 = ubp.reshape(BIAS_ROWS, 128)
    ib2 = ibp.reshape(BIAS_ROWS, 128)
    return _mf_kernel(users, items, uf4, if4, ub2, ib2, global_bias)


# speedup vs baseline: 1.0814x; 1.0814x over previous
import functools

import jax
import jax.numpy as jnp
from jax import lax
from jax.experimental import pallas as pl
from jax.experimental.pallas import tpu as pltpu
from jax.experimental.pallas import tpu_sc as plsc

N_CORES = 2
N_SUBCORES = 16
NW = N_CORES * N_SUBCORES
LANES = 16

B = 16384
D = 32
BPW = B // NW
CHUNK = 128
NCHUNK = BPW // CHUNK
GPC = CHUNK // LANES

UF_ROWS = 250000
BIAS_ROWS = 7813


def _mf_body(users_hbm, items_hbm, uf_hbm, if_hbm, ub_hbm, ib_hbm, gb_hbm,
             out_hbm, uidx_v, iidx_v, urow_v, irow_v, ubrow_v, ibrow_v,
             ufat_v, ifat_v, ubias_v, ibias_v, gb_v, out_v, fsem, bsem):
    wid = lax.axis_index("s") * N_CORES + lax.axis_index("c")
    base = wid * BPW
    pltpu.sync_copy(users_hbm.at[pl.ds(base, BPW)], uidx_v)
    pltpu.sync_copy(items_hbm.at[pl.ds(base, BPW)], iidx_v)

    gb_v[...] = jnp.zeros((LANES,), jnp.float32)
    pltpu.sync_copy(gb_hbm, gb_v.at[pl.ds(0, 1)])

    lanes = lax.iota(jnp.int32, LANES)

    for c in range(NCHUNK):
        csplat = jnp.full((LANES,), c, jnp.int32)
        for k in range(GPC):
            src = lanes + (c * CHUNK + k * LANES)
            dst = lanes + k * LANES
            u = plsc.load_gather(uidx_v, [src])
            i = plsc.load_gather(iidx_v, [src])
            plsc.store_scatter(urow_v, [csplat, dst], u >> 2)
            plsc.store_scatter(irow_v, [csplat, dst], i >> 2)
            plsc.store_scatter(ubrow_v, [csplat, dst], u >> 7)
            plsc.store_scatter(ibrow_v, [csplat, dst], i >> 7)

    gbs = jnp.sum(gb_v[...])

    def fire_factors(c, slot):
        return (
            pltpu.async_copy(uf_hbm.at[urow_v.at[c]], ufat_v.at[slot], fsem),
            pltpu.async_copy(if_hbm.at[irow_v.at[c]], ifat_v.at[slot], fsem),
        )

    def fire_biases(c):
        return (
            pltpu.async_copy(ub_hbm.at[ubrow_v.at[c]], ubias_v, bsem),
            pltpu.async_copy(ib_hbm.at[ibrow_v.at[c]], ibias_v, bsem),
        )

    inflight_f = fire_factors(0, 0)
    inflight_b = fire_biases(0)

    for c in range(NCHUNK):
        for cp in inflight_f:
            cp.wait()
        if c + 1 < NCHUNK:
            next_f = fire_factors(c + 1, (c + 1) % 2)
        else:
            next_f = ()
        for cp in inflight_b:
            cp.wait()
        slot = c % 2
        srow = jnp.full((LANES,), slot, jnp.int32)
        for g in range(GPC):
            e_in_chunk = lanes + g * LANES
            src = e_in_chunk + c * CHUNK
            u = plsc.load_gather(uidx_v, [src])
            i = plsc.load_gather(iidx_v, [src])
            ucol = (u & 3) << 5
            icol = (i & 3) << 5
            acc = (plsc.load_gather(ubias_v, [e_in_chunk, u & 127])
                   + plsc.load_gather(ibias_v, [e_in_chunk, i & 127]) + gbs)
            for j in range(D):
                uv = plsc.load_gather(ufat_v, [srow, e_in_chunk, ucol + j])
                iv = plsc.load_gather(ifat_v, [srow, e_in_chunk, icol + j])
                acc = acc + uv * iv
            plsc.store_scatter(out_v, [src], acc)
        if c + 1 < NCHUNK:
            inflight_b = fire_biases(c + 1)
        inflight_f = next_f

    pltpu.sync_copy(out_v, out_hbm.at[pl.ds(base, BPW)])


@functools.partial(
    pl.kernel,
    out_type=jax.ShapeDtypeStruct((B,), jnp.float32),
    mesh=plsc.VectorSubcoreMesh(core_axis_name="c", subcore_axis_name="s"),
    compiler_params=pltpu.CompilerParams(needs_layout_passes=False),
    scratch_types=[
        pltpu.VMEM((BPW,), jnp.int32),
        pltpu.VMEM((BPW,), jnp.int32),
        pltpu.VMEM((NCHUNK, CHUNK), jnp.int32),
        pltpu.VMEM((NCHUNK, CHUNK), jnp.int32),
        pltpu.VMEM((NCHUNK, CHUNK), jnp.int32),
        pltpu.VMEM((NCHUNK, CHUNK), jnp.int32),
        pltpu.VMEM((2, CHUNK, 128), jnp.float32),
        pltpu.VMEM((2, CHUNK, 128), jnp.float32),
        pltpu.VMEM((CHUNK, 128), jnp.float32),
        pltpu.VMEM((CHUNK, 128), jnp.float32),
        pltpu.VMEM((LANES,), jnp.float32),
        pltpu.VMEM((BPW,), jnp.float32),
        pltpu.SemaphoreType.DMA,
        pltpu.SemaphoreType.DMA,
    ],
)
def _mf_kernel(*refs):
    _mf_body(*refs)


RL_BLOCKS = 7812
RL_PER_W = 489


def _rl_body(ut_hbm, it_hbm, ut_tail_hbm, it_tail_hbm, uf4_hbm, if4_hbm,
             bl0_v, bl1_v, ot0_v, ot1_v,
             is0, is1, os0, os1):
    wid = lax.axis_index("s") * N_CORES + lax.axis_index("c")
    is_user = wid < 16
    k = jnp.where(is_user, wid, wid - 16)
    start = k * RL_PER_W
    end = jnp.minimum(start + RL_PER_W, RL_BLOCKS)
    lanes = lax.iota(jnp.int32, LANES)

    @pl.when(wid == 0)
    def _():
        pltpu.sync_copy(ut_tail_hbm, bl0_v.at[pl.ds(0, 16)])
        pltpu.sync_copy(bl0_v.at[pl.ds(0, 16)], uf4_hbm.at[pl.ds(UF_ROWS - 16, 16)])

    @pl.when(wid == 16)
    def _():
        pltpu.sync_copy(it_tail_hbm, bl0_v.at[pl.ds(0, 16)])
        pltpu.sync_copy(bl0_v.at[pl.ds(0, 16)], if4_hbm.at[pl.ds(UF_ROWS - 16, 16)])

    def run_table(src, dst):
        nbp = (RL_PER_W + 1) // 2

        def clamp(b):
            return jnp.minimum(b, end - 1)

        def fire_in(b, bl, sem):
            col = pl.multiple_of(clamp(b) * 128, 128)
            return pltpu.async_copy(src.at[:, pl.ds(col, 128)], bl, sem)

        def fire_out(b, ot, sem):
            row = pl.multiple_of(clamp(b) * 32, 32)
            return pltpu.async_copy(ot, dst.at[pl.ds(row, 32)], sem)

        fire_in(start, bl0_v, is0)
        fire_in(start + 1, bl1_v, is1)

        def transpose(bl, ot):
            for g in range(GPC):
                e = lanes + g * LANES
                r = e >> 2
                bc = (e & 3) << 5
                for j in range(D):
                    jsp = jnp.full((LANES,), j, jnp.int32)
                    vals = plsc.load_gather(bl, [jsp, e])
                    plsc.store_scatter(ot, [r, bc + jsp], vals)

        def pair(t2, carry):
            b0 = start + 2 * t2
            b1 = b0 + 1
            pltpu.make_async_copy(src.at[:, pl.ds(0, 128)], bl0_v, is0).wait()

            @pl.when(t2 >= 1)
            def _():
                pltpu.make_async_copy(ot0_v, dst.at[pl.ds(0, 32)], os0).wait()

            transpose(bl0_v, ot0_v)
            fire_in(b0 + 2, bl0_v, is0)
            fire_out(b0, ot0_v, os0)
            pltpu.make_async_copy(src.at[:, pl.ds(0, 128)], bl1_v, is1).wait()

            @pl.when(t2 >= 1)
            def _():
                pltpu.make_async_copy(ot1_v, dst.at[pl.ds(0, 32)], os1).wait()

            transpose(bl1_v, ot1_v)
            fire_in(b1 + 2, bl1_v, is1)
            fire_out(b1, ot1_v, os1)
            return carry

        lax.fori_loop(0, nbp, pair, 0)
        pltpu.make_async_copy(src.at[:, pl.ds(0, 128)], bl0_v, is0).wait()
        pltpu.make_async_copy(src.at[:, pl.ds(0, 128)], bl1_v, is1).wait()
        pltpu.make_async_copy(ot0_v, dst.at[pl.ds(0, 32)], os0).wait()
        pltpu.make_async_copy(ot1_v, dst.at[pl.ds(0, 32)], os1).wait()

    @pl.when(is_user)
    def _():
        run_table(ut_hbm, uf4_hbm)

    @pl.when(jnp.logical_not(is_user))
    def _():
        run_table(it_hbm, if4_hbm)


@functools.partial(
    pl.kernel,
    out_type=(jax.ShapeDtypeStruct((UF_ROWS, 128), jnp.float32),
              jax.ShapeDtypeStruct((UF_ROWS, 128), jnp.float32)),
    mesh=plsc.VectorSubcoreMesh(core_axis_name="c", subcore_axis_name="s"),
    compiler_params=pltpu.CompilerParams(needs_layout_passes=False),
    scratch_types=[
        pltpu.VMEM((D, 128), jnp.float32),
        pltpu.VMEM((D, 128), jnp.float32),
        pltpu.VMEM((D, 128), jnp.float32),
        pltpu.VMEM((D, 128), jnp.float32),
        pltpu.SemaphoreType.DMA,
        pltpu.SemaphoreType.DMA,
        pltpu.SemaphoreType.DMA,
        pltpu.SemaphoreType.DMA,
    ],
)
def _rl_kernel(*refs):
    _rl_body(*refs)


def kernel(data, user_factors, item_factors, user_bias, item_bias, global_bias):
    users = data[:, 0]
    items = data[:, 1]
    ut = user_factors.T
    it = item_factors.T
    ut_tail = user_factors[RL_BLOCKS * 128:].reshape(16, 128)
    it_tail = item_factors[RL_BLOCKS * 128:].reshape(16, 128)
    uf4, if4 = _rl_kernel(ut, it, ut_tail, it_tail)
    ubp = jnp.pad(user_bias[:, 0], (0, BIAS_ROWS * 128 - user_bias.shape[0]))
    ibp = jnp.pad(item_bias[:, 0], (0, BIAS_ROWS * 128 - item_bias.shape[0]))
    ub2 = ubp.reshape(BIAS_ROWS, 128)
    ib2 = ibp.reshape(BIAS_ROWS, 128)
    return _mf_kernel(users, items, uf4, if4, ub2, ib2, global_bias)
